# Initial kernel scaffold; baseline (speedup 1.0000x reference)
import jax
import jax.numpy as jnp
from jax.experimental import pallas as pl


def kernel(x, wt, b):
    raise NotImplementedError("write your pallas kernel here")


# speedup vs baseline: 1.9745x; 1.9745x over previous
import jax
import jax.numpy as jnp
from jax.experimental import pallas as pl
from jax.experimental.pallas import tpu as pltpu


def _fused_matmul_kernel(x_ref, w_ref, b_ref, o_ref):
    acc = jnp.dot(x_ref[...], w_ref[...], preferred_element_type=jnp.float32)
    o_ref[...] = jnp.maximum(acc + b_ref[...], 0.0)


def kernel(x, wt, b):
    M, K = x.shape
    Kw, N = wt.shape
    assert Kw == K

    xb = x.astype(jnp.bfloat16)
    wb = wt.astype(jnp.bfloat16)

    TM, TN = 2048, 512
    grid = (M // TM, N // TN)

    return pl.pallas_call(
        _fused_matmul_kernel,
        out_shape=jax.ShapeDtypeStruct((M, N), jnp.float32),
        grid=grid,
        in_specs=[
            pl.BlockSpec((TM, K), lambda i, j: (i, 0)),
            pl.BlockSpec((K, TN), lambda i, j: (0, j)),
            pl.BlockSpec((1, TN), lambda i, j: (0, j)),
        ],
        out_specs=pl.BlockSpec((TM, TN), lambda i, j: (i, j)),
        compiler_params=pltpu.CompilerParams(
            dimension_semantics=("parallel", "parallel"),
            vmem_limit_bytes=56 << 20,
        ),
        cost_estimate=pl.CostEstimate(
            flops=2 * M * N * K,
            transcendentals=0,
            bytes_accessed=M * K * 2 + K * N * 2 + N * 4 + M * N * 4,
        ),
    )(xb, wb, b)


# speedup vs baseline: 2.1902x; 1.1092x over previous
import jax
import jax.numpy as jnp
from jax.experimental import pallas as pl
from jax.experimental.pallas import tpu as pltpu

_TM = 512
_TN = 1024


def _fused_matmul_kernel(x_ref, w_hbm, b_ref, o_ref, w_vmem, sem):
    m = pl.program_id(1)
    j = pl.program_id(2)

    @pl.when(jnp.logical_and(m == 0, j == 0))
    def _():
        cp = pltpu.make_async_copy(w_hbm, w_vmem, sem)
        cp.start()
        cp.wait()

    xb = x_ref[...].astype(jnp.bfloat16)
    acc = jnp.dot(xb, w_vmem[:, pl.ds(j * _TN, _TN)],
                  preferred_element_type=jnp.float32)
    o_ref[...] = jnp.maximum(acc + b_ref[...], 0.0)


def kernel(x, wt, b):
    M, K = x.shape
    Kw, N = wt.shape
    assert Kw == K

    wb = wt.astype(jnp.bfloat16)

    GM, GN = M // _TM, N // _TN
    GMH = GM // 2

    return pl.pallas_call(
        _fused_matmul_kernel,
        out_shape=jax.ShapeDtypeStruct((M, N), jnp.float32),
        grid_spec=pltpu.PrefetchScalarGridSpec(
            num_scalar_prefetch=0,
            grid=(2, GMH, GN),
            in_specs=[
                pl.BlockSpec((_TM, K), lambda c, m, j: (c * GMH + m, 0)),
                pl.BlockSpec(memory_space=pltpu.MemorySpace.HBM),
                pl.BlockSpec((1, _TN), lambda c, m, j: (0, j)),
            ],
            out_specs=pl.BlockSpec((_TM, _TN), lambda c, m, j: (c * GMH + m, j)),
            scratch_shapes=[pltpu.VMEM((K, N), jnp.bfloat16),
                            pltpu.SemaphoreType.DMA],
        ),
        compiler_params=pltpu.CompilerParams(
            dimension_semantics=("parallel", "arbitrary", "arbitrary"),
            vmem_limit_bytes=60 << 20,
        ),
        cost_estimate=pl.CostEstimate(
            flops=2 * M * N * K,
            transcendentals=0,
            bytes_accessed=M * K * 4 + K * N * 2 + N * 4 + M * N * 4,
        ),
    )(x, wb, b)
